# baseline (speedup 1.0000x reference)
import functools

import jax
import jax.numpy as jnp
from jax import lax
from jax.experimental import pallas as pl
from jax.experimental.pallas import tpu as pltpu
from jax.experimental.pallas import tpu_sc as plsc

B = 4096
SEQ = 200
D = 64
HIDDEN = 128
N_CLASSES = 100
S0 = 96
S1 = 104

NC = 2
NS = 16
NW = NC * NS
BPW = B // NW
LANES = 16
DCH = D // LANES


def _make_bagmean():
  mesh = plsc.VectorSubcoreMesh(core_axis_name="c", subcore_axis_name="s")

  @functools.partial(
      pl.kernel,
      mesh=mesh,
      compiler_params=pltpu.CompilerParams(use_tc_tiling_on_sc=False,
                                           needs_layout_passes=False),
      out_type=jax.ShapeDtypeStruct((B, D), jnp.float32),
      scratch_types=[
          pltpu.VMEM((BPW * SEQ,), jnp.int32),
          pltpu.VMEM((2, SEQ, D), jnp.float32),
          pltpu.VMEM((BPW, D), jnp.float32),
          pltpu.SemaphoreType.DMA,
          pltpu.SemaphoreType.DMA,
      ],
  )
  def bagmean(idx_hbm, table_hbm, out_hbm, idx_v, rows_v, out_v,
              sem0, sem1):
    wid = lax.axis_index("s") * NC + lax.axis_index("c")
    base = wid * BPW
    pltpu.sync_copy(idx_hbm.at[pl.ds(base * SEQ, BPW * SEQ)], idx_v)

    sems = (sem0, sem1)

    def issue(j, b):
      pltpu.async_copy(table_hbm.at[idx_v.at[pl.ds(j * SEQ, S0)]],
                       rows_v.at[b, pl.ds(0, S0)], sems[b])
      pltpu.async_copy(table_hbm.at[idx_v.at[pl.ds(j * SEQ + S0, S1)]],
                       rows_v.at[b, pl.ds(S0, S1)], sems[b])

    def wait_pair(j, b):
      pltpu.make_async_copy(table_hbm.at[idx_v.at[pl.ds(j * SEQ, S0)]],
                            rows_v.at[b, pl.ds(0, S0)], sems[b]).wait()
      pltpu.make_async_copy(table_hbm.at[idx_v.at[pl.ds(j * SEQ + S0, S1)]],
                            rows_v.at[b, pl.ds(S0, S1)], sems[b]).wait()

    def accumulate(j, b):
      def row_body(r, acc):
        return tuple(acc[c] + rows_v[b, r, pl.ds(c * LANES, LANES)]
                     for c in range(DCH))

      acc = lax.fori_loop(
          0, SEQ, row_body,
          tuple(jnp.zeros((LANES,), jnp.float32) for _ in range(DCH)),
          unroll=8)
      for c in range(DCH):
        out_v[j, pl.ds(c * LANES, LANES)] = acc[c] * (1.0 / SEQ)

    issue(0, 0)

    def pair_body(p, carry):
      j = 2 * p
      issue(j + 1, 1)
      wait_pair(j, 0)
      accumulate(j, 0)
      issue(lax.rem(j + 2, BPW), 0)
      wait_pair(j + 1, 1)
      accumulate(j + 1, 1)
      return carry

    lax.fori_loop(0, BPW // 2, pair_body, 0)
    wait_pair(0, 0)

    pltpu.sync_copy(out_v, out_hbm.at[pl.ds(base, BPW)])

  return bagmean


def _depad_body(in_ref, out_ref):
  x = in_ref[...]
  y = jnp.reshape(x, (x.shape[0] // 2, 2, x.shape[1]))
  out_ref[...] = jnp.concatenate([y[:, 0, :], y[:, 1, :]], axis=1)


def _depad_table(table):
  blk = 2000
  return pl.pallas_call(
      _depad_body,
      grid=(table.shape[0] // blk,),
      in_specs=[pl.BlockSpec((blk, D), lambda i: (i, 0))],
      out_specs=pl.BlockSpec((blk // 2, 2 * D), lambda i: (i, 0)),
      out_shape=jax.ShapeDtypeStruct((table.shape[0] // 2, 2 * D),
                                     jnp.float32),
  )(table)


def _mlp_body(x_ref, w1t_ref, b1_ref, w2t_ref, b2_ref, out_ref):
  x = x_ref[...]
  h = jnp.dot(x, w1t_ref[...], preferred_element_type=jnp.float32)
  h = jnp.maximum(h + b1_ref[...], 0.0)
  out_ref[...] = (
      jnp.dot(h, w2t_ref[...], preferred_element_type=jnp.float32)
      + b2_ref[...])


def _mlp(x, W1, b1, W2, b2):
  blk = 512
  grid = (B // blk,)
  return pl.pallas_call(
      _mlp_body,
      grid=grid,
      in_specs=[
          pl.BlockSpec((blk, D), lambda i: (i, 0)),
          pl.BlockSpec((D, HIDDEN), lambda i: (0, 0)),
          pl.BlockSpec((1, HIDDEN), lambda i: (0, 0)),
          pl.BlockSpec((HIDDEN, N_CLASSES), lambda i: (0, 0)),
          pl.BlockSpec((1, N_CLASSES), lambda i: (0, 0)),
      ],
      out_specs=pl.BlockSpec((blk, N_CLASSES), lambda i: (i, 0)),
      out_shape=jax.ShapeDtypeStruct((B, N_CLASSES), jnp.float32),
  )(x, W1.T, b1.reshape(1, HIDDEN), W2.T, b2.reshape(1, N_CLASSES))


def kernel(batch_input, table, W1, b1, W2, b2):
  tbl128 = _depad_table(table)
  x = _make_bagmean()(batch_input.reshape(-1),
                      jnp.reshape(tbl128, table.shape))
  return _mlp(x, W1, b1, W2, b2)


# speedup vs baseline: 1.1917x; 1.1917x over previous
import functools

import jax
import jax.numpy as jnp
from jax import lax
from jax.experimental import pallas as pl
from jax.experimental.pallas import tpu as pltpu
from jax.experimental.pallas import tpu_sc as plsc

B = 4096
SEQ = 200
D = 64
HIDDEN = 128
N_CLASSES = 100
S0 = 96
S1 = 104

NC = 2
NS = 16
NW = NC * NS
BPW = B // NW
LANES = 16
DCH = D // LANES


def _make_bagmean():
  mesh = plsc.VectorSubcoreMesh(core_axis_name="c", subcore_axis_name="s")

  @functools.partial(
      pl.kernel,
      mesh=mesh,
      compiler_params=pltpu.CompilerParams(use_tc_tiling_on_sc=False,
                                           needs_layout_passes=False),
      out_type=jax.ShapeDtypeStruct((B, D), jnp.float32),
      scratch_types=[
          pltpu.VMEM((BPW * SEQ,), jnp.int32),
          pltpu.VMEM((2, SEQ, D), jnp.float32),
          pltpu.VMEM((BPW, D), jnp.float32),
          pltpu.SemaphoreType.DMA,
          pltpu.SemaphoreType.DMA,
      ],
  )
  def bagmean(idx_hbm, table_hbm, out_hbm, idx_v, rows_v, out_v,
              sem0, sem1):
    wid = lax.axis_index("s") * NC + lax.axis_index("c")
    base = wid * BPW
    pltpu.sync_copy(idx_hbm.at[pl.ds(base * SEQ, BPW * SEQ)], idx_v)

    sems = (sem0, sem1)

    def issue(j, b):
      pltpu.async_copy(table_hbm.at[idx_v.at[pl.ds(j * SEQ, S0)]],
                       rows_v.at[b, pl.ds(0, S0)], sems[b])
      pltpu.async_copy(table_hbm.at[idx_v.at[pl.ds(j * SEQ + S0, S1)]],
                       rows_v.at[b, pl.ds(S0, S1)], sems[b])

    def wait_pair(j, b):
      pltpu.make_async_copy(table_hbm.at[idx_v.at[pl.ds(j * SEQ, S0)]],
                            rows_v.at[b, pl.ds(0, S0)], sems[b]).wait()
      pltpu.make_async_copy(table_hbm.at[idx_v.at[pl.ds(j * SEQ + S0, S1)]],
                            rows_v.at[b, pl.ds(S0, S1)], sems[b]).wait()

    def accumulate(j, b):
      def row_body(r, acc):
        return tuple(acc[c] + rows_v[b, r, pl.ds(c * LANES, LANES)]
                     for c in range(DCH))

      acc = lax.fori_loop(
          0, SEQ, row_body,
          tuple(jnp.zeros((LANES,), jnp.float32) for _ in range(DCH)),
          unroll=8)
      for c in range(DCH):
        out_v[j, pl.ds(c * LANES, LANES)] = acc[c] * (1.0 / SEQ)

    issue(0, 0)

    def pair_body(p, carry):
      j = 2 * p
      issue(j + 1, 1)
      wait_pair(j, 0)
      accumulate(j, 0)
      issue(lax.rem(j + 2, BPW), 0)
      wait_pair(j + 1, 1)
      accumulate(j + 1, 1)
      return carry

    lax.fori_loop(0, BPW // 2, pair_body, 0)
    wait_pair(0, 0)

    pltpu.sync_copy(out_v, out_hbm.at[pl.ds(base, BPW)])

  return bagmean


def _mlp_body(x_ref, w1t_ref, b1_ref, w2t_ref, b2_ref, out_ref):
  x = x_ref[...]
  h = jnp.dot(x, w1t_ref[...], preferred_element_type=jnp.float32)
  h = jnp.maximum(h + b1_ref[...], 0.0)
  out_ref[...] = (
      jnp.dot(h, w2t_ref[...], preferred_element_type=jnp.float32)
      + b2_ref[...])


def _mlp(x, W1, b1, W2, b2):
  blk = 512
  grid = (B // blk,)
  return pl.pallas_call(
      _mlp_body,
      grid=grid,
      in_specs=[
          pl.BlockSpec((blk, D), lambda i: (i, 0)),
          pl.BlockSpec((D, HIDDEN), lambda i: (0, 0)),
          pl.BlockSpec((1, HIDDEN), lambda i: (0, 0)),
          pl.BlockSpec((HIDDEN, N_CLASSES), lambda i: (0, 0)),
          pl.BlockSpec((1, N_CLASSES), lambda i: (0, 0)),
      ],
      out_specs=pl.BlockSpec((blk, N_CLASSES), lambda i: (i, 0)),
      out_shape=jax.ShapeDtypeStruct((B, N_CLASSES), jnp.float32),
  )(x, W1.T, b1.reshape(1, HIDDEN), W2.T, b2.reshape(1, N_CLASSES))


def kernel(batch_input, table, W1, b1, W2, b2):
  tbl_flat = lax.optimization_barrier(jnp.reshape(table, (-1,)))
  x = _make_bagmean()(batch_input.reshape(-1),
                      jnp.reshape(tbl_flat, table.shape))
  return _mlp(x, W1, b1, W2, b2)


# speedup vs baseline: 1.4560x; 1.2218x over previous
import functools

import jax
import jax.numpy as jnp
from jax import lax
from jax.experimental import pallas as pl
from jax.experimental.pallas import tpu as pltpu
from jax.experimental.pallas import tpu_sc as plsc

B = 4096
SEQ = 200
D = 64
HIDDEN = 128
N_CLASSES = 100
S0 = 96
S1 = 104

NC = 2
NS = 16
NW = NC * NS
BPW = B // NW
LANES = 16
DCH = D // LANES


def _make_bagmean():
  mesh = plsc.VectorSubcoreMesh(core_axis_name="c", subcore_axis_name="s")

  @functools.partial(
      pl.kernel,
      mesh=mesh,
      compiler_params=pltpu.CompilerParams(use_tc_tiling_on_sc=False,
                                           needs_layout_passes=False),
      out_type=jax.ShapeDtypeStruct((B, D), jnp.float32),
      scratch_types=[
          pltpu.VMEM((BPW * SEQ,), jnp.int32),
          pltpu.VMEM((4, SEQ, D), jnp.float32),
          pltpu.VMEM((BPW, D), jnp.float32),
          pltpu.SemaphoreType.DMA,
          pltpu.SemaphoreType.DMA,
          pltpu.SemaphoreType.DMA,
          pltpu.SemaphoreType.DMA,
      ],
  )
  def bagmean(idx_hbm, table_hbm, out_hbm, idx_v, rows_v, out_v,
              sem0, sem1, sem2, sem3):
    wid = lax.axis_index("s") * NC + lax.axis_index("c")
    base = wid * BPW
    pltpu.sync_copy(idx_hbm.at[pl.ds(base * SEQ, BPW * SEQ)], idx_v)

    sems = (sem0, sem1, sem2, sem3)

    def issue(j, b):
      pltpu.async_copy(table_hbm.at[idx_v.at[pl.ds(j * SEQ, S0)]],
                       rows_v.at[b, pl.ds(0, S0)], sems[b])
      pltpu.async_copy(table_hbm.at[idx_v.at[pl.ds(j * SEQ + S0, S1)]],
                       rows_v.at[b, pl.ds(S0, S1)], sems[b])

    def wait_pair(j, b):
      pltpu.make_async_copy(table_hbm.at[idx_v.at[pl.ds(j * SEQ, S0)]],
                            rows_v.at[b, pl.ds(0, S0)], sems[b]).wait()
      pltpu.make_async_copy(table_hbm.at[idx_v.at[pl.ds(j * SEQ + S0, S1)]],
                            rows_v.at[b, pl.ds(S0, S1)], sems[b]).wait()

    def accumulate(j, b):
      def row_body(r, acc):
        return tuple(acc[c] + rows_v[b, r, pl.ds(c * LANES, LANES)]
                     for c in range(DCH))

      acc = lax.fori_loop(
          0, SEQ, row_body,
          tuple(jnp.zeros((LANES,), jnp.float32) for _ in range(DCH)),
          unroll=8)
      for c in range(DCH):
        out_v[j, pl.ds(c * LANES, LANES)] = acc[c] * (1.0 / SEQ)

    for b in range(4):
      issue(b, b)

    def quad_body(p, carry):
      j0 = 4 * p
      for b in range(4):
        j = j0 + b
        wait_pair(j, b)
        accumulate(j, b)
        issue(lax.rem(j + 4, BPW), b)
      return carry

    lax.fori_loop(0, BPW // 4, quad_body, 0)
    for b in range(4):
      wait_pair(b, b)

    pltpu.sync_copy(out_v, out_hbm.at[pl.ds(base, BPW)])

  return bagmean


def _mlp_body(x_ref, w1t_ref, b1_ref, w2t_ref, b2_ref, out_ref):
  x = x_ref[...]
  h = jnp.dot(x, w1t_ref[...], preferred_element_type=jnp.float32)
  h = jnp.maximum(h + b1_ref[...], 0.0)
  out_ref[...] = (
      jnp.dot(h, w2t_ref[...], preferred_element_type=jnp.float32)
      + b2_ref[...])


def _mlp(x, W1, b1, W2, b2):
  blk = 512
  grid = (B // blk,)
  return pl.pallas_call(
      _mlp_body,
      grid=grid,
      in_specs=[
          pl.BlockSpec((blk, D), lambda i: (i, 0)),
          pl.BlockSpec((D, HIDDEN), lambda i: (0, 0)),
          pl.BlockSpec((1, HIDDEN), lambda i: (0, 0)),
          pl.BlockSpec((HIDDEN, N_CLASSES), lambda i: (0, 0)),
          pl.BlockSpec((1, N_CLASSES), lambda i: (0, 0)),
      ],
      out_specs=pl.BlockSpec((blk, N_CLASSES), lambda i: (i, 0)),
      out_shape=jax.ShapeDtypeStruct((B, N_CLASSES), jnp.float32),
  )(x, W1.T, b1.reshape(1, HIDDEN), W2.T, b2.reshape(1, N_CLASSES))


def kernel(batch_input, table, W1, b1, W2, b2):
  tbl_flat = lax.optimization_barrier(jnp.reshape(table, (-1,)))
  x = _make_bagmean()(batch_input.reshape(-1),
                      jnp.reshape(tbl_flat, table.shape))
  return _mlp(x, W1, b1, W2, b2)
